# Initial kernel scaffold; baseline (speedup 1.0000x reference)
import jax
import jax.numpy as jnp
from jax.experimental import pallas as pl


def kernel(q, v, embeddings):
    raise NotImplementedError("write your pallas kernel here")


# speedup vs baseline: 8.1694x; 8.1694x over previous
import functools

import jax
import jax.numpy as jnp
from jax import lax
from jax.experimental import pallas as pl
from jax.experimental.pallas import tpu as pltpu
from jax.experimental.pallas import tpu_sc as plsc


@functools.lru_cache(maxsize=None)
def _make_rel_pos_kernel(q_len, v_len, in_dim, out_dim):
    info = plsc.get_sparse_core_info()
    nc, ns = info.num_cores, info.num_subcores
    nw = nc * ns

    mp = (in_dim - 1) // 2
    pre = q_len - 1 - mp
    suf_start = pre + in_dim
    r_len = q_len + v_len - 1
    assert suf_start + (v_len - 1 - mp) == r_len
    assert v_len - 1 - mp == pre
    assert q_len % nw == 0 and out_dim % 16 == 0
    rows_per_w = q_len // nw
    row_w = v_len * out_dim

    mesh = plsc.VectorSubcoreMesh(core_axis_name="c", subcore_axis_name="s")

    @functools.partial(
        pl.kernel,
        out_type=jax.ShapeDtypeStruct((q_len * row_w,), jnp.float32),
        mesh=mesh,
        scratch_types=[
            pltpu.VMEM((r_len * out_dim,), jnp.float32),
            pltpu.SemaphoreType.DMA,
        ],
    )
    def rel_pos(emb_hbm, out_hbm, r_v, sem):
        wid = lax.axis_index("s") * nc + lax.axis_index("c")

        pltpu.sync_copy(emb_hbm, r_v.at[pl.ds(pre * out_dim, in_dim * out_dim)])
        nchunk = out_dim // 16
        first = [r_v[pl.ds(pre * out_dim + c * 16, 16)] for c in range(nchunk)]
        last = [r_v[pl.ds((suf_start - 1) * out_dim + c * 16, 16)]
                for c in range(nchunk)]

        def fill(t, carry):
            base0 = t * out_dim
            base1 = (suf_start + t) * out_dim
            for c in range(nchunk):
                r_v[pl.ds(base0 + c * 16, 16)] = first[c]
                r_v[pl.ds(base1 + c * 16, 16)] = last[c]
            return carry

        lax.fori_loop(0, pre, fill, 0)

        base = wid * rows_per_w
        handles = []
        for r in range(rows_per_w):
            i = base + r
            src = r_v.at[pl.ds((q_len - 1 - i) * out_dim, row_w)]
            dst = out_hbm.at[pl.ds(i * row_w, row_w)]
            handles.append(pltpu.async_copy(src, dst, sem))
        for h in handles:
            h.wait()

    return rel_pos


def kernel(q, v, embeddings):
    q_len = q.shape[1]
    v_len = v.shape[1]
    in_dim, out_dim = embeddings.shape
    rel_pos = _make_rel_pos_kernel(q_len, v_len, in_dim, out_dim)
    out = rel_pos(embeddings.reshape(-1))
    return out.reshape(q_len, v_len, out_dim)
